# baseline (speedup 1.0000x reference)
import functools

import jax
import jax.numpy as jnp
from jax.experimental import pallas as pl
from jax.experimental.pallas import tpu as pltpu

ALPHA_ = 0.1
BM = 512
BK = 512


def _prop_kernel(adj_ref, x_ref, h_ref, o_ref, acc_ref, *, nk):
    k = pl.program_id(1)

    @pl.when(k == 0)
    def _init():
        acc_ref[...] = jnp.zeros_like(acc_ref)

    acc_ref[...] += jnp.dot(
        adj_ref[...], x_ref[...], preferred_element_type=jnp.float32
    )

    @pl.when(k == nk - 1)
    def _epilogue():
        o_ref[...] = (1.0 - ALPHA_) * acc_ref[...] + ALPHA_ * h_ref[...]


@jax.jit
def kernel(x, adj, h):
    n, d = x.shape
    nm = n // BM
    nk = n // BK
    return pl.pallas_call(
        functools.partial(_prop_kernel, nk=nk),
        grid=(nm, nk),
        in_specs=[
            pl.BlockSpec((BM, BK), lambda i, k: (i, k)),
            pl.BlockSpec((BK, d), lambda i, k: (k, 0)),
            pl.BlockSpec((BM, d), lambda i, k: (i, 0)),
        ],
        out_specs=pl.BlockSpec((BM, d), lambda i, k: (i, 0)),
        out_shape=jax.ShapeDtypeStruct((n, d), jnp.float32),
        scratch_shapes=[pltpu.VMEM((BM, d), jnp.float32)],
        compiler_params=pltpu.CompilerParams(
            dimension_semantics=("parallel", "arbitrary"),
        ),
    )(adj, x, h)


# speedup vs baseline: 2.2169x; 2.2169x over previous
import functools

import jax
import jax.numpy as jnp
from jax.experimental import pallas as pl
from jax.experimental.pallas import tpu as pltpu

ALPHA_ = 0.1
BM = 512


def _prop_kernel(adj_ref, x_ref, h_ref, o_ref):
    o_ref[...] = (1.0 - ALPHA_) * jnp.dot(
        adj_ref[...], x_ref[...], preferred_element_type=jnp.float32
    ) + ALPHA_ * h_ref[...]


@jax.jit
def kernel(x, adj, h):
    n, d = x.shape
    nm = n // BM
    return pl.pallas_call(
        _prop_kernel,
        grid=(nm,),
        in_specs=[
            pl.BlockSpec((BM, n), lambda i: (i, 0)),
            pl.BlockSpec((n, d), lambda i: (0, 0)),
            pl.BlockSpec((BM, d), lambda i: (i, 0)),
        ],
        out_specs=pl.BlockSpec((BM, d), lambda i: (i, 0)),
        out_shape=jax.ShapeDtypeStruct((n, d), jnp.float32),
        compiler_params=pltpu.CompilerParams(
            dimension_semantics=("parallel",),
        ),
    )(adj, x, h)
